# baseline (speedup 1.0000x reference)
import functools

import jax
import jax.numpy as jnp
from jax import lax
from jax.experimental import pallas as pl
from jax.experimental.pallas import tpu as pltpu
from jax.experimental.pallas import tpu_sc as plsc

N = 10000
NPAD = 10240
E = 320000
F = 8
FA = 5
NC = 2
NS = 16
NW = NC * NS
EPW = E // NW
CHUNK = 2000
NBUF = 3
NCHUNK = EPW // CHUNK
GPC = CHUNK // 16
DSLICE = NPAD // NS
NRED = 10
RW = NPAD * FA // NRED
RSUB = RW // NS
RW8 = RW * F // FA
RSUB8 = RSUB * F // FA

_mesh = plsc.VectorSubcoreMesh(core_axis_name="c", subcore_axis_name="s")
f32 = jnp.float32
i32 = jnp.int32


def _deg_body(dst_hbm, out_hbm, dst_v, cnt_v, res_v, res8_v, slab):
    c = lax.axis_index("c")
    s = lax.axis_index("s")
    wid = c * NS + s
    pltpu.sync_copy(dst_hbm.at[pl.ds(wid * EPW, EPW)], dst_v)

    zeros16 = jnp.zeros((16,), f32)

    def zbody(i, carry):
        for u in range(16):
            cnt_v[pl.ds((i * 16 + u) * 16, 16)] = zeros16
        return carry

    lax.fori_loop(0, NPAD // 256, zbody, 0)

    ones16 = jnp.ones((16,), f32)

    def body(i, carry):
        for u in range(5):
            d = dst_v[pl.ds((i * 5 + u) * 16, 16)]
            plsc.addupdate_scatter(cnt_v, [d], ones16)
        return carry

    lax.fori_loop(0, EPW // 80, body, 0)

    pltpu.sync_copy(cnt_v, slab.at[s])
    plsc.subcore_barrier()
    for t in range(NS):
        pltpu.sync_copy(slab.at[t, pl.ds(s * DSLICE, DSLICE)],
                        cnt_v.at[pl.ds(t * DSLICE, DSLICE)])

    def sumbody(j, carry):
        tot = cnt_v[pl.ds(j * 16, 16)]
        for t in range(1, NS):
            tot = tot + cnt_v[pl.ds(t * DSLICE + j * 16, 16)]
        res_v[pl.ds(j * 16, 16)] = tot
        return carry

    lax.fori_loop(0, DSLICE // 16, sumbody, 0)

    iota = lax.iota(i32, 16)
    rep = iota >> 3

    def expbody(j, carry):
        v = plsc.load_gather(res_v, [j * 2 + rep])
        res8_v[pl.ds(j * 16, 16)] = v
        return carry

    lax.fori_loop(0, DSLICE * F // 16, expbody, 0)
    pltpu.sync_copy(res8_v, out_hbm.at[c, pl.ds(s * DSLICE * F, DSLICE * F)])


_sc_params = pltpu.CompilerParams(needs_layout_passes=False,
                                  use_tc_tiling_on_sc=False)

_deg = functools.partial(
    pl.kernel,
    out_type=jax.ShapeDtypeStruct((NC, NPAD * F), f32),
    mesh=_mesh,
    compiler_params=_sc_params,
    scratch_types=[
        pltpu.VMEM((EPW,), i32),
        pltpu.VMEM((NPAD,), f32),
        pltpu.VMEM((DSLICE,), f32),
        pltpu.VMEM((DSLICE * F,), f32),
        pltpu.VMEM_SHARED((NS, NPAD), f32),
    ],
)(_deg_body)


def _agg_body(src_hbm, dst_hbm, g_hbm, out_hbm, acc_v, stage_v, res_v, res8_v,
              r0, r1, r2, sb0, sb1, sb2, d0, d1, d2,
              g0, g1, g2, u0, u1, u2, t0, t1, t2, msem, slab):
    c = lax.axis_index("c")
    s = lax.axis_index("s")
    wid = c * NS + s
    ebase = wid * EPW

    rows = [r0, r1, r2]
    srcs = [sb0, sb1, sb2]
    dsts = [d0, d1, d2]
    gsem = [g0, g1, g2]
    ssem = [u0, u1, u2]
    dsem = [t0, t1, t2]

    def src_dma(k):
        return pltpu.async_copy(
            src_hbm.at[pl.ds(ebase + k * CHUNK, CHUNK)],
            srcs[k % NBUF], ssem[k % NBUF])

    def dst_dma(k):
        return pltpu.async_copy(
            dst_hbm.at[pl.ds(ebase + k * CHUNK, CHUNK)],
            dsts[k % NBUF], dsem[k % NBUF])

    def gather_dma(k):
        return pltpu.async_copy(
            g_hbm.at[srcs[k % NBUF]], rows[k % NBUF], gsem[k % NBUF])

    sdescs = {k: src_dma(k) for k in range(NBUF)}
    ddescs = {k: dst_dma(k) for k in range(NBUF)}

    zeros16 = jnp.zeros((16,), f32)

    def zbody(i, carry):
        for u in range(32):
            acc_v[pl.ds((i * 32 + u) * 16, 16)] = zeros16
        return carry

    lax.fori_loop(0, NPAD * FA // 512, zbody, 0)

    gdescs = {}
    for k in range(NBUF):
        sdescs[k].wait()
        gdescs[k] = gather_dma(k)

    iota = lax.iota(i32, 16)
    rot5 = [(iota + f) % FA for f in range(FA)]

    for k in range(NCHUNK):
        gdescs[k].wait()
        ddescs[k].wait()
        if k + NBUF < NCHUNK:
            sdescs[k + NBUF] = src_dma(k + NBUF)
        rb = rows[k % NBUF]
        db = dsts[k % NBUF]

        def proc(it, pc, rb=rb, db=db):
            for u in range(5):
                jj = it * 5 + u
                dvec = db[pl.ds(jj * 16, 16)]
                dbase = dvec * FA
                evec = jj * 16 + iota
                for f in range(FA):
                    val = plsc.load_gather(rb, [evec, rot5[f]])
                    plsc.addupdate_scatter(acc_v, [dbase + rot5[f]], val)
            return pc

        lax.fori_loop(0, GPC // 5, proc, 0)
        if k + NBUF < NCHUNK:
            sdescs[k + NBUF].wait()
            gdescs[k + NBUF] = gather_dma(k + NBUF)
            ddescs[k + NBUF] = dst_dma(k + NBUF)

    exp_idx = (iota >> 3) * FA + (iota & 7)
    exp_msk = (iota & 7) < FA
    for r in range(NRED):
        pltpu.sync_copy(acc_v.at[pl.ds(r * RW, RW)], slab.at[s])
        plsc.subcore_barrier()
        rdescs = [pltpu.async_copy(slab.at[t, pl.ds(s * RSUB, RSUB)],
                                   stage_v.at[pl.ds(t * RSUB, RSUB)], msem)
                  for t in range(NS)]
        for dsc in rdescs:
            dsc.wait()

        def sumbody(j, carry):
            tot = stage_v[pl.ds(j * 16, 16)]
            for t in range(1, NS):
                tot = tot + stage_v[pl.ds(t * RSUB + j * 16, 16)]
            res_v[pl.ds(j * 16, 16)] = tot
            return carry

        lax.fori_loop(0, RSUB // 16, sumbody, 0)

        def expbody(j, carry):
            idx = j * 10 + exp_idx
            v = plsc.load_gather(res_v, [idx])
            res8_v[pl.ds(j * 16, 16)] = jnp.where(exp_msk, v, 0.0)
            return carry

        lax.fori_loop(0, RSUB8 // 16, expbody, 0)
        pltpu.sync_copy(
            res8_v, out_hbm.at[c, pl.ds(r * RW8 + s * RSUB8, RSUB8)])
        plsc.subcore_barrier()


_agg = functools.partial(
    pl.kernel,
    out_type=jax.ShapeDtypeStruct((NC, NPAD * F), f32),
    mesh=_mesh,
    compiler_params=_sc_params,
    scratch_types=[
        pltpu.VMEM((NPAD * FA,), f32),
        pltpu.VMEM((NS * RSUB,), f32),
        pltpu.VMEM((RSUB + 16,), f32),
        pltpu.VMEM((RSUB8,), f32),
    ] + [pltpu.VMEM((CHUNK, F), f32) for _ in range(NBUF)]
      + [pltpu.VMEM((CHUNK,), i32) for _ in range(NBUF)]
      + [pltpu.VMEM((CHUNK,), i32) for _ in range(NBUF)]
      + [pltpu.SemaphoreType.DMA for _ in range(3 * NBUF + 1)]
      + [pltpu.VMEM_SHARED((NS, RW), f32)],
)(_agg_body)


NROW = NPAD // 16
NXR = N // 16


def _lin1_body(x_ref, w_ref, degx_ref, g1_ref, dinvx_ref):
    deg = degx_ref[0] + degx_ref[1]
    r_i = lax.broadcasted_iota(i32, (NROW, 128), 0)
    c_i = lax.broadcasted_iota(i32, (NROW, 128), 1)
    node = r_i * 16 + (c_i >> 3)
    dinvx = jnp.where(node < N, lax.rsqrt(deg + 1.0), 0.0)
    dinvx_ref[...] = dinvx
    h1 = jnp.dot(x_ref[...].reshape(NXR, 16 * 128), w_ref[...],
                 preferred_element_type=f32)
    g1_ref[0:NXR, :] = dinvx[0:NXR, :] * h1
    g1_ref[NXR:NROW, :] = jnp.zeros((NROW - NXR, 128), f32)


_lin1 = pl.pallas_call(
    _lin1_body,
    out_shape=[jax.ShapeDtypeStruct((NROW, 128), f32),
               jax.ShapeDtypeStruct((NROW, 128), f32)],
)


def _mid_body(ax_ref, g_ref, dinvx_ref, w_ref, b_ref, gn_ref):
    dinvx = dinvx_ref[...]
    z = jnp.maximum(
        dinvx * (ax_ref[0] + ax_ref[1] + g_ref[...]) + b_ref[...], 0.0)
    h = jnp.dot(z, w_ref[...], preferred_element_type=f32)
    gn_ref[...] = dinvx * h


_lin2 = pl.pallas_call(
    _mid_body,
    out_shape=jax.ShapeDtypeStruct((NROW, 128), f32),
)


def _fin_body(ax_ref, g_ref, dinvx_ref, w_ref, b_ref, b3_ref, out_ref):
    dinvx = dinvx_ref[...]
    z = jnp.maximum(
        dinvx * (ax_ref[0] + ax_ref[1] + g_ref[...]) + b_ref[...], 0.0)
    out_ref[...] = jnp.dot(z, w_ref[...],
                           preferred_element_type=f32) + b3_ref[...]


_lin3 = pl.pallas_call(
    _fin_body,
    out_shape=jax.ShapeDtypeStruct((NROW, 128), f32),
)


def kernel(x, edge_index, W1, b1, W2, b2, W3, b3):
    src = edge_index[0]
    dst = edge_index[1]
    eye16 = jnp.eye(16, dtype=f32)
    W1p = jnp.zeros((128, F), f32).at[:, :FA].set(W1)
    W1bd = jnp.kron(eye16, W1p)
    W2p = jnp.zeros((F, F), f32).at[:FA, :FA].set(W2)
    W2bd = jnp.kron(eye16, W2p)
    W3p = jnp.zeros((F, F), f32).at[:FA, :7].set(W3)
    W3bd = jnp.kron(eye16, W3p)
    b1x = jnp.tile(jnp.pad(b1, (0, F - FA)), 16).reshape(1, 128)
    b2x = jnp.tile(jnp.pad(b2, (0, F - FA)), 16).reshape(1, 128)
    b3x = jnp.tile(jnp.pad(b3, (0, F - 7)), 16).reshape(1, 128)
    degp = _deg(dst)
    g1, dinvx = _lin1(x, W1bd, degp.reshape(NC, NROW, 128))

    acc1 = _agg(src, dst, g1.reshape(NPAD, F))
    g2 = _lin2(acc1.reshape(NC, NROW, 128), g1, dinvx, W2bd, b1x)

    acc2 = _agg(src, dst, g2.reshape(NPAD, F))
    outp = _lin3(acc2.reshape(NC, NROW, 128), g2, dinvx, W3bd, b2x, b3x)
    return outp.reshape(NPAD, F)[:N, :7]


# speedup vs baseline: 1.0939x; 1.0939x over previous
import functools

import jax
import jax.numpy as jnp
from jax import lax
from jax.experimental import pallas as pl
from jax.experimental.pallas import tpu as pltpu
from jax.experimental.pallas import tpu_sc as plsc

N = 10000
NPAD = 10240
E = 320000
F = 8
FA = 5
NC = 2
NS = 16
NW = NC * NS
EPW = E // NW
CHUNK = 2000
NBUF = 3
NCHUNK = EPW // CHUNK
GPC = CHUNK // 16
DSLICE = NPAD // NS
NRED = 10
RW = NPAD * FA // NRED
RSUB = RW // NS
RW8 = RW * F // FA
RSUB8 = RSUB * F // FA

_mesh = plsc.VectorSubcoreMesh(core_axis_name="c", subcore_axis_name="s")
f32 = jnp.float32
i32 = jnp.int32


def _deg_body(dst_hbm, out_hbm, dst_v, cnt_v, res_v, res8_v, slab):
    c = lax.axis_index("c")
    s = lax.axis_index("s")
    wid = c * NS + s
    pltpu.sync_copy(dst_hbm.at[pl.ds(wid * EPW, EPW)], dst_v)

    zeros16 = jnp.zeros((16,), f32)

    def zbody(i, carry):
        for u in range(16):
            cnt_v[pl.ds((i * 16 + u) * 16, 16)] = zeros16
        return carry

    lax.fori_loop(0, NPAD // 256, zbody, 0)

    ones16 = jnp.ones((16,), f32)

    def body(i, carry):
        for u in range(5):
            d = dst_v[pl.ds((i * 5 + u) * 16, 16)]
            plsc.addupdate_scatter(cnt_v, [d], ones16)
        return carry

    lax.fori_loop(0, EPW // 80, body, 0)

    pltpu.sync_copy(cnt_v, slab.at[s])
    plsc.subcore_barrier()
    for t in range(NS):
        pltpu.sync_copy(slab.at[t, pl.ds(s * DSLICE, DSLICE)],
                        cnt_v.at[pl.ds(t * DSLICE, DSLICE)])

    def sumbody(j, carry):
        tot = cnt_v[pl.ds(j * 16, 16)]
        for t in range(1, NS):
            tot = tot + cnt_v[pl.ds(t * DSLICE + j * 16, 16)]
        res_v[pl.ds(j * 16, 16)] = tot
        return carry

    lax.fori_loop(0, DSLICE // 16, sumbody, 0)

    iota = lax.iota(i32, 16)
    rep = iota >> 3

    def expbody(j, carry):
        v = plsc.load_gather(res_v, [j * 2 + rep])
        res8_v[pl.ds(j * 16, 16)] = v
        return carry

    lax.fori_loop(0, DSLICE * F // 16, expbody, 0)
    pltpu.sync_copy(res8_v, out_hbm.at[c, pl.ds(s * DSLICE * F, DSLICE * F)])


_sc_params = pltpu.CompilerParams(needs_layout_passes=False,
                                  use_tc_tiling_on_sc=False)

_deg = functools.partial(
    pl.kernel,
    out_type=jax.ShapeDtypeStruct((NC, NPAD * F), f32),
    mesh=_mesh,
    compiler_params=_sc_params,
    scratch_types=[
        pltpu.VMEM((EPW,), i32),
        pltpu.VMEM((NPAD,), f32),
        pltpu.VMEM((DSLICE,), f32),
        pltpu.VMEM((DSLICE * F,), f32),
        pltpu.VMEM_SHARED((NS, NPAD), f32),
    ],
)(_deg_body)


def _agg_body(src_hbm, dst_hbm, g_hbm, out_hbm, acc_v, stage_v, res_v, res8_v,
              r0, r1, r2, sb0, sb1, sb2, d0, d1, d2,
              g0, g1, g2, u0, u1, u2, t0, t1, t2, msem, slab):
    c = lax.axis_index("c")
    s = lax.axis_index("s")
    wid = c * NS + s
    ebase = wid * EPW

    rows = [r0, r1, r2]
    srcs = [sb0, sb1, sb2]
    dsts = [d0, d1, d2]
    gsem = [g0, g1, g2]
    ssem = [u0, u1, u2]
    dsem = [t0, t1, t2]

    def src_dma(k):
        return pltpu.async_copy(
            src_hbm.at[pl.ds(ebase + k * CHUNK, CHUNK)],
            srcs[k % NBUF], ssem[k % NBUF])

    def dst_dma(k):
        return pltpu.async_copy(
            dst_hbm.at[pl.ds(ebase + k * CHUNK, CHUNK)],
            dsts[k % NBUF], dsem[k % NBUF])

    def gather_dma(k):
        return pltpu.async_copy(
            g_hbm.at[srcs[k % NBUF]], rows[k % NBUF], gsem[k % NBUF])

    sdescs = {k: src_dma(k) for k in range(NBUF)}
    ddescs = {k: dst_dma(k) for k in range(NBUF)}

    zeros16 = jnp.zeros((16,), f32)

    def zbody(i, carry):
        for u in range(32):
            acc_v[pl.ds((i * 32 + u) * 16, 16)] = zeros16
        return carry

    lax.fori_loop(0, NPAD * FA // 512, zbody, 0)

    gdescs = {}
    for k in range(NBUF):
        sdescs[k].wait()
        gdescs[k] = gather_dma(k)

    iota = lax.iota(i32, 16)
    rot5 = [(iota + f) % FA for f in range(FA)]

    for k in range(NCHUNK):
        gdescs[k].wait()
        ddescs[k].wait()
        if k + NBUF < NCHUNK:
            sdescs[k + NBUF] = src_dma(k + NBUF)
        rb = rows[k % NBUF]
        db = dsts[k % NBUF]

        def proc(it, pc, rb=rb, db=db):
            for u in range(5):
                jj = it * 5 + u
                dvec = db[pl.ds(jj * 16, 16)]
                dbase = dvec * FA
                evec = jj * 16 + iota
                for f in range(FA):
                    val = plsc.load_gather(rb, [evec, rot5[f]])
                    plsc.addupdate_scatter(acc_v, [dbase + rot5[f]], val)
            return pc

        lax.fori_loop(0, GPC // 5, proc, 0)
        if k + NBUF < NCHUNK:
            sdescs[k + NBUF].wait()
            gdescs[k + NBUF] = gather_dma(k + NBUF)
            ddescs[k + NBUF] = dst_dma(k + NBUF)

    exp_idx = (iota >> 3) * FA + (iota & 7)
    exp_msk = (iota & 7) < FA
    for r in range(NRED):
        pltpu.sync_copy(acc_v.at[pl.ds(r * RW, RW)], slab.at[s])
        plsc.subcore_barrier()
        rdescs = [pltpu.async_copy(slab.at[t, pl.ds(s * RSUB, RSUB)],
                                   stage_v.at[pl.ds(t * RSUB, RSUB)], msem)
                  for t in range(NS)]
        for dsc in rdescs:
            dsc.wait()

        def sumbody(j, carry):
            tot = stage_v[pl.ds(j * 16, 16)]
            for t in range(1, NS):
                tot = tot + stage_v[pl.ds(t * RSUB + j * 16, 16)]
            res_v[pl.ds(j * 16, 16)] = tot
            return carry

        lax.fori_loop(0, RSUB // 16, sumbody, 0)

        def expbody(j, carry):
            idx = j * 10 + exp_idx
            v = plsc.load_gather(res_v, [idx])
            res8_v[pl.ds(j * 16, 16)] = jnp.where(exp_msk, v, 0.0)
            return carry

        lax.fori_loop(0, RSUB8 // 16, expbody, 0)
        pltpu.sync_copy(
            res8_v, out_hbm.at[c, pl.ds(r * RW8 + s * RSUB8, RSUB8)])
        plsc.subcore_barrier()


_agg = functools.partial(
    pl.kernel,
    out_type=jax.ShapeDtypeStruct((NC, NPAD * F), f32),
    mesh=_mesh,
    compiler_params=_sc_params,
    scratch_types=[
        pltpu.VMEM((NPAD * FA,), f32),
        pltpu.VMEM((NS * RSUB,), f32),
        pltpu.VMEM((RSUB + 16,), f32),
        pltpu.VMEM((RSUB8,), f32),
    ] + [pltpu.VMEM((CHUNK, F), f32) for _ in range(NBUF)]
      + [pltpu.VMEM((CHUNK,), i32) for _ in range(NBUF)]
      + [pltpu.VMEM((CHUNK,), i32) for _ in range(NBUF)]
      + [pltpu.SemaphoreType.DMA for _ in range(3 * NBUF + 1)]
      + [pltpu.VMEM_SHARED((NS, RW), f32)],
)(_agg_body)


NROW = NPAD // 16
NXR = N // 16


def _edges_body(ei_ref, src_ref, dst_ref):
    src_ref[...] = ei_ref[0, :]
    dst_ref[...] = ei_ref[1, :]


_edges = pl.pallas_call(
    _edges_body,
    out_shape=[jax.ShapeDtypeStruct((E,), i32),
               jax.ShapeDtypeStruct((E,), i32)],
)


def _mm1_body(x_ref, w_ref, h1_ref):
    h1_ref[...] = jnp.dot(x_ref[...].reshape(NXR, 16 * 128), w_ref[...],
                          preferred_element_type=f32)


_mm1 = pl.pallas_call(
    _mm1_body,
    out_shape=jax.ShapeDtypeStruct((NXR, 128), f32),
)


def _lin1_body(h1_ref, degx_ref, g1_ref, dinvx_ref):
    deg = degx_ref[0] + degx_ref[1]
    r_i = lax.broadcasted_iota(i32, (NROW, 128), 0)
    c_i = lax.broadcasted_iota(i32, (NROW, 128), 1)
    node = r_i * 16 + (c_i >> 3)
    dinvx = jnp.where(node < N, lax.rsqrt(deg + 1.0), 0.0)
    dinvx_ref[...] = dinvx
    g1_ref[0:NXR, :] = dinvx[0:NXR, :] * h1_ref[...]
    g1_ref[NXR:NROW, :] = jnp.zeros((NROW - NXR, 128), f32)


_lin1 = pl.pallas_call(
    _lin1_body,
    out_shape=[jax.ShapeDtypeStruct((NROW, 128), f32),
               jax.ShapeDtypeStruct((NROW, 128), f32)],
)


def _mid_body(ax_ref, g_ref, dinvx_ref, w_ref, b_ref, gn_ref):
    dinvx = dinvx_ref[...]
    z = jnp.maximum(
        dinvx * (ax_ref[0] + ax_ref[1] + g_ref[...]) + b_ref[...], 0.0)
    h = jnp.dot(z, w_ref[...], preferred_element_type=f32)
    gn_ref[...] = dinvx * h


_lin2 = pl.pallas_call(
    _mid_body,
    out_shape=jax.ShapeDtypeStruct((NROW, 128), f32),
)


def _fin_body(ax_ref, g_ref, dinvx_ref, w_ref, b_ref, b3_ref, out_ref):
    dinvx = dinvx_ref[...]
    z = jnp.maximum(
        dinvx * (ax_ref[0] + ax_ref[1] + g_ref[...]) + b_ref[...], 0.0)
    out_ref[...] = jnp.dot(z, w_ref[...],
                           preferred_element_type=f32) + b3_ref[...]


_lin3 = pl.pallas_call(
    _fin_body,
    out_shape=jax.ShapeDtypeStruct((NROW, 128), f32),
)


def kernel(x, edge_index, W1, b1, W2, b2, W3, b3):
    src, dst = _edges(edge_index)
    eye16 = jnp.eye(16, dtype=f32)
    W1p = jnp.zeros((128, F), f32).at[:, :FA].set(W1)
    W1bd = jnp.kron(eye16, W1p)
    W2p = jnp.zeros((F, F), f32).at[:FA, :FA].set(W2)
    W2bd = jnp.kron(eye16, W2p)
    W3p = jnp.zeros((F, F), f32).at[:FA, :7].set(W3)
    W3bd = jnp.kron(eye16, W3p)
    b1x = jnp.tile(jnp.pad(b1, (0, F - FA)), 16).reshape(1, 128)
    b2x = jnp.tile(jnp.pad(b2, (0, F - FA)), 16).reshape(1, 128)
    b3x = jnp.tile(jnp.pad(b3, (0, F - 7)), 16).reshape(1, 128)
    h1 = _mm1(x, W1bd)
    degp = _deg(dst)
    g1, dinvx = _lin1(h1, degp.reshape(NC, NROW, 128))

    acc1 = _agg(src, dst, g1.reshape(NPAD, F))
    g2 = _lin2(acc1.reshape(NC, NROW, 128), g1, dinvx, W2bd, b1x)

    acc2 = _agg(src, dst, g2.reshape(NPAD, F))
    outp = _lin3(acc2.reshape(NC, NROW, 128), g2, dinvx, W3bd, b2x, b3x)
    return outp.reshape(NPAD, F)[:N, :7]


# speedup vs baseline: 1.2182x; 1.1137x over previous
import functools

import jax
import jax.numpy as jnp
from jax import lax
from jax.experimental import pallas as pl
from jax.experimental.pallas import tpu as pltpu
from jax.experimental.pallas import tpu_sc as plsc

N = 10000
NPAD = 10240
E = 320000
F = 8
FA = 5
NC = 2
NS = 16
NW = NC * NS
EPW = E // NW
CHUNK = 2000
NBUF = 3
NCHUNK = EPW // CHUNK
GPC = CHUNK // 16
DSLICE = NPAD // NS
NRED = 10
RW = NPAD * FA // NRED
RSUB = RW // NS
RW8 = RW * F // FA
RSUB8 = RSUB * F // FA

_mesh = plsc.VectorSubcoreMesh(core_axis_name="c", subcore_axis_name="s")
f32 = jnp.float32
i32 = jnp.int32


def _deg_body(dst_hbm, out_hbm, dst_v, cnt_v, res_v, res8_v, slab):
    c = lax.axis_index("c")
    s = lax.axis_index("s")
    wid = c * NS + s
    pltpu.sync_copy(dst_hbm.at[pl.ds(wid * EPW, EPW)], dst_v)

    zeros16 = jnp.zeros((16,), f32)

    def zbody(i, carry):
        for u in range(16):
            cnt_v[pl.ds((i * 16 + u) * 16, 16)] = zeros16
        return carry

    lax.fori_loop(0, NPAD // 256, zbody, 0)

    ones16 = jnp.ones((16,), f32)

    def body(i, carry):
        for u in range(5):
            d = dst_v[pl.ds((i * 5 + u) * 16, 16)]
            plsc.addupdate_scatter(cnt_v, [d], ones16)
        return carry

    lax.fori_loop(0, EPW // 80, body, 0)

    pltpu.sync_copy(cnt_v, slab.at[s])
    plsc.subcore_barrier()
    for t in range(NS):
        pltpu.sync_copy(slab.at[t, pl.ds(s * DSLICE, DSLICE)],
                        cnt_v.at[pl.ds(t * DSLICE, DSLICE)])

    def sumbody(j, carry):
        tot = cnt_v[pl.ds(j * 16, 16)]
        for t in range(1, NS):
            tot = tot + cnt_v[pl.ds(t * DSLICE + j * 16, 16)]
        res_v[pl.ds(j * 16, 16)] = tot
        return carry

    lax.fori_loop(0, DSLICE // 16, sumbody, 0)

    iota = lax.iota(i32, 16)
    rep = iota >> 3

    def expbody(j, carry):
        v = plsc.load_gather(res_v, [j * 2 + rep])
        res8_v[pl.ds(j * 16, 16)] = v
        return carry

    lax.fori_loop(0, DSLICE * F // 16, expbody, 0)
    pltpu.sync_copy(res8_v, out_hbm.at[c, pl.ds(s * DSLICE * F, DSLICE * F)])


_sc_params = pltpu.CompilerParams(needs_layout_passes=False,
                                  use_tc_tiling_on_sc=False)

_deg = functools.partial(
    pl.kernel,
    out_type=jax.ShapeDtypeStruct((NC, NPAD * F), f32),
    mesh=_mesh,
    compiler_params=_sc_params,
    scratch_types=[
        pltpu.VMEM((EPW,), i32),
        pltpu.VMEM((NPAD,), f32),
        pltpu.VMEM((DSLICE,), f32),
        pltpu.VMEM((DSLICE * F,), f32),
        pltpu.VMEM_SHARED((NS, NPAD), f32),
    ],
)(_deg_body)


def _agg_body(src_hbm, dst_hbm, g_hbm, out_hbm, acc_v, stage_v, res_v, res8_v,
              r0, r1, r2, sb0, sb1, sb2, d0, d1, d2,
              g0, g1, g2, u0, u1, u2, t0, t1, t2, msem, slab):
    c = lax.axis_index("c")
    s = lax.axis_index("s")
    wid = c * NS + s
    ebase = wid * EPW

    rows = [r0, r1, r2]
    srcs = [sb0, sb1, sb2]
    dsts = [d0, d1, d2]
    gsem = [g0, g1, g2]
    ssem = [u0, u1, u2]
    dsem = [t0, t1, t2]

    def src_dma(k):
        return pltpu.async_copy(
            src_hbm.at[pl.ds(ebase + k * CHUNK, CHUNK)],
            srcs[k % NBUF], ssem[k % NBUF])

    def dst_dma(k):
        return pltpu.async_copy(
            dst_hbm.at[pl.ds(ebase + k * CHUNK, CHUNK)],
            dsts[k % NBUF], dsem[k % NBUF])

    def gather_dma(k):
        return pltpu.async_copy(
            g_hbm.at[srcs[k % NBUF]], rows[k % NBUF], gsem[k % NBUF])

    sdescs = {k: src_dma(k) for k in range(NBUF)}
    ddescs = {k: dst_dma(k) for k in range(NBUF)}

    zeros16 = jnp.zeros((16,), f32)

    def zbody(i, carry):
        for u in range(32):
            acc_v[pl.ds((i * 32 + u) * 16, 16)] = zeros16
        return carry

    lax.fori_loop(0, NPAD * FA // 512, zbody, 0)

    gdescs = {}
    for k in range(NBUF):
        sdescs[k].wait()
        gdescs[k] = gather_dma(k)

    iota = lax.iota(i32, 16)
    rot5 = [(iota + f) % FA for f in range(FA)]

    for k in range(NCHUNK):
        gdescs[k].wait()
        ddescs[k].wait()
        if k + NBUF < NCHUNK:
            sdescs[k + NBUF] = src_dma(k + NBUF)
        rb = rows[k % NBUF]
        db = dsts[k % NBUF]

        @plsc.parallel_loop(0, GPC, unroll=5)
        def _(jj, rb=rb, db=db):
            dvec = db[pl.ds(jj * 16, 16)]
            dbase = dvec * FA
            evec = jj * 16 + iota
            for f in range(FA):
                val = plsc.load_gather(rb, [evec, rot5[f]])
                plsc.addupdate_scatter(acc_v, [dbase + rot5[f]], val)
        if k + NBUF < NCHUNK:
            sdescs[k + NBUF].wait()
            gdescs[k + NBUF] = gather_dma(k + NBUF)
            ddescs[k + NBUF] = dst_dma(k + NBUF)

    exp_idx = (iota >> 3) * FA + (iota & 7)
    exp_msk = (iota & 7) < FA
    for r in range(NRED):
        pltpu.sync_copy(acc_v.at[pl.ds(r * RW, RW)], slab.at[s])
        plsc.subcore_barrier()
        rdescs = [pltpu.async_copy(slab.at[t, pl.ds(s * RSUB, RSUB)],
                                   stage_v.at[pl.ds(t * RSUB, RSUB)], msem)
                  for t in range(NS)]
        for dsc in rdescs:
            dsc.wait()

        def sumbody(j, carry):
            tot = stage_v[pl.ds(j * 16, 16)]
            for t in range(1, NS):
                tot = tot + stage_v[pl.ds(t * RSUB + j * 16, 16)]
            res_v[pl.ds(j * 16, 16)] = tot
            return carry

        lax.fori_loop(0, RSUB // 16, sumbody, 0)

        def expbody(j, carry):
            idx = j * 10 + exp_idx
            v = plsc.load_gather(res_v, [idx])
            res8_v[pl.ds(j * 16, 16)] = jnp.where(exp_msk, v, 0.0)
            return carry

        lax.fori_loop(0, RSUB8 // 16, expbody, 0)
        pltpu.sync_copy(
            res8_v, out_hbm.at[c, pl.ds(r * RW8 + s * RSUB8, RSUB8)])
        plsc.subcore_barrier()


_agg = functools.partial(
    pl.kernel,
    out_type=jax.ShapeDtypeStruct((NC, NPAD * F), f32),
    mesh=_mesh,
    compiler_params=_sc_params,
    scratch_types=[
        pltpu.VMEM((NPAD * FA,), f32),
        pltpu.VMEM((NS * RSUB,), f32),
        pltpu.VMEM((RSUB + 16,), f32),
        pltpu.VMEM((RSUB8,), f32),
    ] + [pltpu.VMEM((CHUNK, F), f32) for _ in range(NBUF)]
      + [pltpu.VMEM((CHUNK,), i32) for _ in range(NBUF)]
      + [pltpu.VMEM((CHUNK,), i32) for _ in range(NBUF)]
      + [pltpu.SemaphoreType.DMA for _ in range(3 * NBUF + 1)]
      + [pltpu.VMEM_SHARED((NS, RW), f32)],
)(_agg_body)


NROW = NPAD // 16
NXR = N // 16


def _edges_body(ei_ref, src_ref, dst_ref):
    src_ref[...] = ei_ref[0, :]
    dst_ref[...] = ei_ref[1, :]


_edges = pl.pallas_call(
    _edges_body,
    out_shape=[jax.ShapeDtypeStruct((E,), i32),
               jax.ShapeDtypeStruct((E,), i32)],
)


def _mm1_body(x_ref, w_ref, h1_ref):
    h1_ref[...] = jnp.dot(x_ref[...].reshape(NXR, 16 * 128), w_ref[...],
                          preferred_element_type=f32)


_mm1 = pl.pallas_call(
    _mm1_body,
    out_shape=jax.ShapeDtypeStruct((NXR, 128), f32),
)


def _lin1_body(h1_ref, degx_ref, g1_ref, dinvx_ref):
    deg = degx_ref[0] + degx_ref[1]
    r_i = lax.broadcasted_iota(i32, (NROW, 128), 0)
    c_i = lax.broadcasted_iota(i32, (NROW, 128), 1)
    node = r_i * 16 + (c_i >> 3)
    dinvx = jnp.where(node < N, lax.rsqrt(deg + 1.0), 0.0)
    dinvx_ref[...] = dinvx
    g1_ref[0:NXR, :] = dinvx[0:NXR, :] * h1_ref[...]
    g1_ref[NXR:NROW, :] = jnp.zeros((NROW - NXR, 128), f32)


_lin1 = pl.pallas_call(
    _lin1_body,
    out_shape=[jax.ShapeDtypeStruct((NROW, 128), f32),
               jax.ShapeDtypeStruct((NROW, 128), f32)],
)


def _mid_body(ax_ref, g_ref, dinvx_ref, w_ref, b_ref, gn_ref):
    dinvx = dinvx_ref[...]
    z = jnp.maximum(
        dinvx * (ax_ref[0] + ax_ref[1] + g_ref[...]) + b_ref[...], 0.0)
    h = jnp.dot(z, w_ref[...], preferred_element_type=f32)
    gn_ref[...] = dinvx * h


_lin2 = pl.pallas_call(
    _mid_body,
    out_shape=jax.ShapeDtypeStruct((NROW, 128), f32),
)


def _fin_body(ax_ref, g_ref, dinvx_ref, w_ref, b_ref, b3_ref, out_ref):
    dinvx = dinvx_ref[...]
    z = jnp.maximum(
        dinvx * (ax_ref[0] + ax_ref[1] + g_ref[...]) + b_ref[...], 0.0)
    out_ref[...] = jnp.dot(z, w_ref[...],
                           preferred_element_type=f32) + b3_ref[...]


_lin3 = pl.pallas_call(
    _fin_body,
    out_shape=jax.ShapeDtypeStruct((NROW, 128), f32),
)


def kernel(x, edge_index, W1, b1, W2, b2, W3, b3):
    src, dst = _edges(edge_index)
    eye16 = jnp.eye(16, dtype=f32)
    W1p = jnp.zeros((128, F), f32).at[:, :FA].set(W1)
    W1bd = jnp.kron(eye16, W1p)
    W2p = jnp.zeros((F, F), f32).at[:FA, :FA].set(W2)
    W2bd = jnp.kron(eye16, W2p)
    W3p = jnp.zeros((F, F), f32).at[:FA, :7].set(W3)
    W3bd = jnp.kron(eye16, W3p)
    b1x = jnp.tile(jnp.pad(b1, (0, F - FA)), 16).reshape(1, 128)
    b2x = jnp.tile(jnp.pad(b2, (0, F - FA)), 16).reshape(1, 128)
    b3x = jnp.tile(jnp.pad(b3, (0, F - 7)), 16).reshape(1, 128)
    h1 = _mm1(x, W1bd)
    degp = _deg(dst)
    g1, dinvx = _lin1(h1, degp.reshape(NC, NROW, 128))

    acc1 = _agg(src, dst, g1.reshape(NPAD, F))
    g2 = _lin2(acc1.reshape(NC, NROW, 128), g1, dinvx, W2bd, b1x)

    acc2 = _agg(src, dst, g2.reshape(NPAD, F))
    outp = _lin3(acc2.reshape(NC, NROW, 128), g2, dinvx, W3bd, b2x, b3x)
    return outp.reshape(NPAD, F)[:N, :7]


# speedup vs baseline: 1.2471x; 1.0237x over previous
import functools

import jax
import jax.numpy as jnp
from jax import lax
from jax.experimental import pallas as pl
from jax.experimental.pallas import tpu as pltpu
from jax.experimental.pallas import tpu_sc as plsc

N = 10000
NPAD = 10240
E = 320000
F = 8
FA = 5
NC = 2
NS = 16
NW = NC * NS
EPW = E // NW
CHUNK = 2000
NBUF = 3
NCHUNK = EPW // CHUNK
GPC = CHUNK // 16
DSLICE = NPAD // NS
NRED = 10
RW = NPAD * FA // NRED
RSUB = RW // NS
RW8 = RW * F // FA
RSUB8 = RSUB * F // FA

_mesh = plsc.VectorSubcoreMesh(core_axis_name="c", subcore_axis_name="s")
f32 = jnp.float32
i32 = jnp.int32


def _deg_body(dst_hbm, out_hbm, dst_v, cnt_v, res_v, res8_v, slab):
    c = lax.axis_index("c")
    s = lax.axis_index("s")
    wid = c * NS + s
    pltpu.sync_copy(dst_hbm.at[pl.ds(wid * EPW, EPW)], dst_v)

    zeros16 = jnp.zeros((16,), f32)

    @plsc.parallel_loop(0, NPAD // 16, unroll=8)
    def _(i):
        cnt_v[pl.ds(i * 16, 16)] = zeros16

    ones16 = jnp.ones((16,), f32)

    @plsc.parallel_loop(0, EPW // 16, unroll=8)
    def _(i):
        d = dst_v[pl.ds(i * 16, 16)]
        plsc.addupdate_scatter(cnt_v, [d], ones16)

    pltpu.sync_copy(cnt_v, slab.at[s])
    plsc.subcore_barrier()
    for t in range(NS):
        pltpu.sync_copy(slab.at[t, pl.ds(s * DSLICE, DSLICE)],
                        cnt_v.at[pl.ds(t * DSLICE, DSLICE)])

    @plsc.parallel_loop(0, DSLICE // 16, unroll=4)
    def _(j):
        tot = cnt_v[pl.ds(j * 16, 16)]
        for t in range(1, NS):
            tot = tot + cnt_v[pl.ds(t * DSLICE + j * 16, 16)]
        res_v[pl.ds(j * 16, 16)] = tot

    iota = lax.iota(i32, 16)
    rep = iota >> 3

    @plsc.parallel_loop(0, DSLICE * F // 16, unroll=8)
    def _(j):
        v = plsc.load_gather(res_v, [j * 2 + rep])
        res8_v[pl.ds(j * 16, 16)] = v
    pltpu.sync_copy(res8_v, out_hbm.at[c, pl.ds(s * DSLICE * F, DSLICE * F)])


_sc_params = pltpu.CompilerParams(needs_layout_passes=False,
                                  use_tc_tiling_on_sc=False)

_deg = functools.partial(
    pl.kernel,
    out_type=jax.ShapeDtypeStruct((NC, NPAD * F), f32),
    mesh=_mesh,
    compiler_params=_sc_params,
    scratch_types=[
        pltpu.VMEM((EPW,), i32),
        pltpu.VMEM((NPAD,), f32),
        pltpu.VMEM((DSLICE,), f32),
        pltpu.VMEM((DSLICE * F,), f32),
        pltpu.VMEM_SHARED((NS, NPAD), f32),
    ],
)(_deg_body)


def _agg_body(src_hbm, dst_hbm, g_hbm, out_hbm, acc_v, stage_v, res_v, res8_v,
              r0, r1, r2, sb0, sb1, sb2, d0, d1, d2,
              g0, g1, g2, u0, u1, u2, t0, t1, t2, msem, slab):
    c = lax.axis_index("c")
    s = lax.axis_index("s")
    wid = c * NS + s
    ebase = wid * EPW

    rows = [r0, r1, r2]
    srcs = [sb0, sb1, sb2]
    dsts = [d0, d1, d2]
    gsem = [g0, g1, g2]
    ssem = [u0, u1, u2]
    dsem = [t0, t1, t2]

    def src_dma(k):
        return pltpu.async_copy(
            src_hbm.at[pl.ds(ebase + k * CHUNK, CHUNK)],
            srcs[k % NBUF], ssem[k % NBUF])

    def dst_dma(k):
        return pltpu.async_copy(
            dst_hbm.at[pl.ds(ebase + k * CHUNK, CHUNK)],
            dsts[k % NBUF], dsem[k % NBUF])

    def gather_dma(k):
        return pltpu.async_copy(
            g_hbm.at[srcs[k % NBUF]], rows[k % NBUF], gsem[k % NBUF])

    sdescs = {k: src_dma(k) for k in range(NBUF)}
    ddescs = {k: dst_dma(k) for k in range(NBUF)}

    zeros16 = jnp.zeros((16,), f32)

    @plsc.parallel_loop(0, NPAD * FA // 16, unroll=8)
    def _(i):
        acc_v[pl.ds(i * 16, 16)] = zeros16

    gdescs = {}
    for k in range(NBUF):
        sdescs[k].wait()
        gdescs[k] = gather_dma(k)

    iota = lax.iota(i32, 16)
    rot5 = [(iota + f) % FA for f in range(FA)]

    for k in range(NCHUNK):
        gdescs[k].wait()
        ddescs[k].wait()
        if k + NBUF < NCHUNK:
            sdescs[k + NBUF] = src_dma(k + NBUF)
        rb = rows[k % NBUF]
        db = dsts[k % NBUF]

        @plsc.parallel_loop(0, GPC, unroll=5)
        def _(jj, rb=rb, db=db):
            dvec = db[pl.ds(jj * 16, 16)]
            dbase = dvec * FA
            evec = jj * 16 + iota
            for f in range(FA):
                val = plsc.load_gather(rb, [evec, rot5[f]])
                plsc.addupdate_scatter(acc_v, [dbase + rot5[f]], val)
        if k + NBUF < NCHUNK:
            sdescs[k + NBUF].wait()
            gdescs[k + NBUF] = gather_dma(k + NBUF)
            ddescs[k + NBUF] = dst_dma(k + NBUF)

    exp_idx = (iota >> 3) * FA + (iota & 7)
    exp_msk = (iota & 7) < FA
    for r in range(NRED):
        pltpu.sync_copy(acc_v.at[pl.ds(r * RW, RW)], slab.at[s])
        plsc.subcore_barrier()
        rdescs = [pltpu.async_copy(slab.at[t, pl.ds(s * RSUB, RSUB)],
                                   stage_v.at[pl.ds(t * RSUB, RSUB)], msem)
                  for t in range(NS)]
        for dsc in rdescs:
            dsc.wait()

        @plsc.parallel_loop(0, RSUB // 16, unroll=4)
        def _(j):
            tot = stage_v[pl.ds(j * 16, 16)]
            for t in range(1, NS):
                tot = tot + stage_v[pl.ds(t * RSUB + j * 16, 16)]
            res_v[pl.ds(j * 16, 16)] = tot

        @plsc.parallel_loop(0, RSUB8 // 16, unroll=8)
        def _(j):
            idx = j * 10 + exp_idx
            v = plsc.load_gather(res_v, [idx])
            res8_v[pl.ds(j * 16, 16)] = jnp.where(exp_msk, v, 0.0)
        pltpu.sync_copy(
            res8_v, out_hbm.at[c, pl.ds(r * RW8 + s * RSUB8, RSUB8)])
        plsc.subcore_barrier()


_agg = functools.partial(
    pl.kernel,
    out_type=jax.ShapeDtypeStruct((NC, NPAD * F), f32),
    mesh=_mesh,
    compiler_params=_sc_params,
    scratch_types=[
        pltpu.VMEM((NPAD * FA,), f32),
        pltpu.VMEM((NS * RSUB,), f32),
        pltpu.VMEM((RSUB + 16,), f32),
        pltpu.VMEM((RSUB8,), f32),
    ] + [pltpu.VMEM((CHUNK, F), f32) for _ in range(NBUF)]
      + [pltpu.VMEM((CHUNK,), i32) for _ in range(NBUF)]
      + [pltpu.VMEM((CHUNK,), i32) for _ in range(NBUF)]
      + [pltpu.SemaphoreType.DMA for _ in range(3 * NBUF + 1)]
      + [pltpu.VMEM_SHARED((NS, RW), f32)],
)(_agg_body)


NROW = NPAD // 16
NXR = N // 16


def _edges_body(ei_ref, src_ref, dst_ref):
    src_ref[...] = ei_ref[0, :]
    dst_ref[...] = ei_ref[1, :]


_edges = pl.pallas_call(
    _edges_body,
    out_shape=[jax.ShapeDtypeStruct((E,), i32),
               jax.ShapeDtypeStruct((E,), i32)],
)


def _mm1_body(x_ref, w_ref, h1_ref):
    h1_ref[...] = jnp.dot(x_ref[...].reshape(NXR, 16 * 128), w_ref[...],
                          preferred_element_type=f32)


_mm1 = pl.pallas_call(
    _mm1_body,
    out_shape=jax.ShapeDtypeStruct((NXR, 128), f32),
)


def _lin1_body(h1_ref, degx_ref, g1_ref, dinvx_ref):
    deg = degx_ref[0] + degx_ref[1]
    r_i = lax.broadcasted_iota(i32, (NROW, 128), 0)
    c_i = lax.broadcasted_iota(i32, (NROW, 128), 1)
    node = r_i * 16 + (c_i >> 3)
    dinvx = jnp.where(node < N, lax.rsqrt(deg + 1.0), 0.0)
    dinvx_ref[...] = dinvx
    g1_ref[0:NXR, :] = dinvx[0:NXR, :] * h1_ref[...]
    g1_ref[NXR:NROW, :] = jnp.zeros((NROW - NXR, 128), f32)


_lin1 = pl.pallas_call(
    _lin1_body,
    out_shape=[jax.ShapeDtypeStruct((NROW, 128), f32),
               jax.ShapeDtypeStruct((NROW, 128), f32)],
)


def _mid_body(ax_ref, g_ref, dinvx_ref, w_ref, b_ref, gn_ref):
    dinvx = dinvx_ref[...]
    z = jnp.maximum(
        dinvx * (ax_ref[0] + ax_ref[1] + g_ref[...]) + b_ref[...], 0.0)
    h = jnp.dot(z, w_ref[...], preferred_element_type=f32)
    gn_ref[...] = dinvx * h


_lin2 = pl.pallas_call(
    _mid_body,
    out_shape=jax.ShapeDtypeStruct((NROW, 128), f32),
)


def _fin_body(ax_ref, g_ref, dinvx_ref, w_ref, b_ref, b3_ref, out_ref):
    dinvx = dinvx_ref[...]
    z = jnp.maximum(
        dinvx * (ax_ref[0] + ax_ref[1] + g_ref[...]) + b_ref[...], 0.0)
    out_ref[...] = jnp.dot(z, w_ref[...],
                           preferred_element_type=f32) + b3_ref[...]


_lin3 = pl.pallas_call(
    _fin_body,
    out_shape=jax.ShapeDtypeStruct((NROW, 128), f32),
)


def kernel(x, edge_index, W1, b1, W2, b2, W3, b3):
    src, dst = _edges(edge_index)
    eye16 = jnp.eye(16, dtype=f32)
    W1p = jnp.zeros((128, F), f32).at[:, :FA].set(W1)
    W1bd = jnp.kron(eye16, W1p)
    W2p = jnp.zeros((F, F), f32).at[:FA, :FA].set(W2)
    W2bd = jnp.kron(eye16, W2p)
    W3p = jnp.zeros((F, F), f32).at[:FA, :7].set(W3)
    W3bd = jnp.kron(eye16, W3p)
    b1x = jnp.tile(jnp.pad(b1, (0, F - FA)), 16).reshape(1, 128)
    b2x = jnp.tile(jnp.pad(b2, (0, F - FA)), 16).reshape(1, 128)
    b3x = jnp.tile(jnp.pad(b3, (0, F - 7)), 16).reshape(1, 128)
    h1 = _mm1(x, W1bd)
    degp = _deg(dst)
    g1, dinvx = _lin1(h1, degp.reshape(NC, NROW, 128))

    acc1 = _agg(src, dst, g1.reshape(NPAD, F))
    g2 = _lin2(acc1.reshape(NC, NROW, 128), g1, dinvx, W2bd, b1x)

    acc2 = _agg(src, dst, g2.reshape(NPAD, F))
    outp = _lin3(acc2.reshape(NC, NROW, 128), g2, dinvx, W3bd, b2x, b3x)
    return outp.reshape(NPAD, F)[:N, :7]
